# Initial kernel scaffold; baseline (speedup 1.0000x reference)
import jax
import jax.numpy as jnp
from jax.experimental import pallas as pl


def kernel(x, gate_up_proj, down_proj):
    raise NotImplementedError("write your pallas kernel here")


# speedup vs baseline: 1.1028x; 1.1028x over previous
import jax
import jax.numpy as jnp
from jax.experimental import pallas as pl


def _mlp_block(x_ref, gup_ref, dp_ref, o_ref):
    i = dp_ref.shape[1]
    gu = jnp.dot(x_ref[...], gup_ref[0], preferred_element_type=jnp.float32)
    gate = gu[:, :i]
    up = gu[:, i:]
    inter = jax.nn.silu(gate) * up
    o_ref[...] = jnp.dot(inter, dp_ref[0], preferred_element_type=jnp.float32)


def kernel(x, gate_up_proj, down_proj):
    b, n, h = x.shape
    e, _, i2 = gate_up_proj.shape
    i = down_proj.shape[1]
    g = n // e
    rows = b * g
    x3 = x.reshape(rows, e * h)
    out3 = pl.pallas_call(
        _mlp_block,
        grid=(e,),
        in_specs=[
            pl.BlockSpec((rows, h), lambda ei: (0, ei)),
            pl.BlockSpec((1, h, i2), lambda ei: (ei, 0, 0)),
            pl.BlockSpec((1, i, h), lambda ei: (ei, 0, 0)),
        ],
        out_specs=pl.BlockSpec((rows, h), lambda ei: (0, ei)),
        out_shape=jax.ShapeDtypeStruct((rows, e * h), jnp.float32),
    )(x3, gate_up_proj, down_proj)
    return out3.reshape(b, n, h)


# speedup vs baseline: 1.1058x; 1.0028x over previous
import jax
import jax.numpy as jnp
from jax.experimental import pallas as pl


def _mlp_block(x_ref, gup_ref, dp_ref, o_ref):
    i = dp_ref.shape[1]
    xb = x_ref[...].astype(jnp.bfloat16)
    gu = jnp.dot(xb, gup_ref[0].astype(jnp.bfloat16),
                 preferred_element_type=jnp.float32)
    gate = gu[:, :i]
    up = gu[:, i:]
    inter = (jax.nn.silu(gate) * up).astype(jnp.bfloat16)
    o_ref[...] = jnp.dot(inter, dp_ref[0].astype(jnp.bfloat16),
                         preferred_element_type=jnp.float32)


def kernel(x, gate_up_proj, down_proj):
    b, n, h = x.shape
    e, _, i2 = gate_up_proj.shape
    i = down_proj.shape[1]
    g = n // e
    rows = b * g
    x3 = x.reshape(rows, e * h)
    out3 = pl.pallas_call(
        _mlp_block,
        grid=(e,),
        in_specs=[
            pl.BlockSpec((rows, h), lambda ei: (0, ei)),
            pl.BlockSpec((1, h, i2), lambda ei: (ei, 0, 0)),
            pl.BlockSpec((1, i, h), lambda ei: (ei, 0, 0)),
        ],
        out_specs=pl.BlockSpec((rows, h), lambda ei: (0, ei)),
        out_shape=jax.ShapeDtypeStruct((rows, e * h), jnp.float32),
    )(x3, gate_up_proj, down_proj)
    return out3.reshape(b, n, h)
